# baseline (speedup 1.0000x reference)
import jax
import jax.numpy as jnp
from jax import lax
from jax.experimental import pallas as pl
from jax.experimental.pallas import tpu as pltpu

NUM_CODES = 1024
DIM = 64
PIX = 1024
BATCH = 8
LOSS_SCALE = 1.25 / (BATCH * PIX * DIM)


def _vq_kernel(z_ref, e_ref, zq_ref, idx_ref, part_ref):
    zb = z_ref[0]
    e = e_ref[...]

    en = jnp.sum(e * e, axis=1)
    zn = jnp.sum(zb * zb, axis=0)

    m_t = lax.dot_general(
        e, zb, (((1,), (0,)), ((), ())),
        preferred_element_type=jnp.float32,
        precision=lax.Precision.DEFAULT,
    )
    dist_t = (zn[None, :] - 2.0 * m_t) + en[:, None]

    md = jnp.min(dist_t, axis=0)
    code_iota = lax.broadcasted_iota(jnp.int32, (NUM_CODES, PIX), 0)
    idx = jnp.min(
        jnp.where(dist_t == md[None, :], code_iota, NUM_CODES), axis=0
    ).astype(jnp.int32)
    idx_ref[0, 0, :] = idx

    one_hot = (code_iota == idx[None, :]).astype(jnp.bfloat16)
    e_hi = e.astype(jnp.bfloat16)
    e_lo = (e - e_hi.astype(jnp.float32)).astype(jnp.bfloat16)
    dn = (((0,), (0,)), ((), ()))
    zq_t = lax.dot_general(
        e_hi, one_hot, dn, preferred_element_type=jnp.float32
    ) + lax.dot_general(
        e_lo, one_hot, dn, preferred_element_type=jnp.float32
    )

    zq_ref[0] = zb + (zq_t - zb)

    part_ref[0, 0, :] = jnp.broadcast_to(jnp.sum(md), (128,))


def kernel(z, embed_w):
    z3 = z.reshape(BATCH, DIM, PIX)
    zq3, idx3, parts = pl.pallas_call(
        _vq_kernel,
        grid=(BATCH,),
        in_specs=[
            pl.BlockSpec((1, DIM, PIX), lambda b: (b, 0, 0)),
            pl.BlockSpec((NUM_CODES, DIM), lambda b: (0, 0)),
        ],
        out_specs=[
            pl.BlockSpec((1, DIM, PIX), lambda b: (b, 0, 0)),
            pl.BlockSpec((1, 1, PIX), lambda b: (b, 0, 0)),
            pl.BlockSpec((1, 1, 128), lambda b: (b, 0, 0)),
        ],
        out_shape=[
            jax.ShapeDtypeStruct((BATCH, DIM, PIX), jnp.float32),
            jax.ShapeDtypeStruct((BATCH, 1, PIX), jnp.int32),
            jax.ShapeDtypeStruct((BATCH, 1, 128), jnp.float32),
        ],
        compiler_params=pltpu.CompilerParams(
            dimension_semantics=(pltpu.PARALLEL,),
        ),
    )(z3, embed_w)
    z_q_st = zq3.reshape(z.shape)
    encoding_indices = idx3.reshape(BATCH, 32, 32)
    loss = jnp.sum(parts[:, 0, 0]) * LOSS_SCALE
    return z_q_st, loss, encoding_indices


# speedup vs baseline: 1.1818x; 1.1818x over previous
import jax
import jax.numpy as jnp
from jax import lax
from jax.experimental import pallas as pl
from jax.experimental.pallas import tpu as pltpu

NUM_CODES = 1024
DIM = 64
PIX = 1024
BATCH = 8
LOSS_SCALE = 1.25 / (BATCH * PIX * DIM)


def _vq_kernel(z_ref, e_ref, zq_ref, idx_ref, loss_ref):
    zb = z_ref[0]
    e = e_ref[...]

    en = jnp.sum(e * e, axis=1)
    zn = jnp.sum(zb * zb, axis=0)

    m_t = lax.dot_general(
        e, zb, (((1,), (0,)), ((), ())),
        preferred_element_type=jnp.float32,
        precision=lax.Precision.DEFAULT,
    )
    dist_t = (zn[None, :] - 2.0 * m_t) + en[:, None]

    md = jnp.min(dist_t, axis=0)
    code_iota = lax.broadcasted_iota(jnp.int32, (NUM_CODES, PIX), 0)
    idx = jnp.min(
        jnp.where(dist_t == md[None, :], code_iota, NUM_CODES), axis=0
    ).astype(jnp.int32)
    idx_ref[0, 0, :] = idx

    one_hot = (code_iota == idx[None, :]).astype(jnp.bfloat16)
    e_hi = e.astype(jnp.bfloat16)
    e_lo = (e - e_hi.astype(jnp.float32)).astype(jnp.bfloat16)
    e_cat = jnp.concatenate([e_hi, e_lo], axis=1)
    zq2 = lax.dot_general(
        e_cat, one_hot, (((0,), (0,)), ((), ())),
        preferred_element_type=jnp.float32,
    )
    zq_t = zq2[:DIM] + zq2[DIM:]

    zq_ref[0] = zb + (zq_t - zb)

    part = jnp.sum(md).reshape(1, 1)
    b = pl.program_id(0)

    @pl.when(b == 0)
    def _():
        loss_ref[...] = jnp.zeros((1, 1), jnp.float32)

    loss_ref[...] += part

    @pl.when(b == BATCH - 1)
    def _():
        loss_ref[...] = loss_ref[...] * LOSS_SCALE


def kernel(z, embed_w):
    z3 = z.reshape(BATCH, DIM, PIX)
    zq3, idx3, loss = pl.pallas_call(
        _vq_kernel,
        grid=(BATCH,),
        in_specs=[
            pl.BlockSpec((1, DIM, PIX), lambda b: (b, 0, 0)),
            pl.BlockSpec((NUM_CODES, DIM), lambda b: (0, 0)),
        ],
        out_specs=[
            pl.BlockSpec((1, DIM, PIX), lambda b: (b, 0, 0)),
            pl.BlockSpec((1, 1, PIX), lambda b: (b, 0, 0)),
            pl.BlockSpec((1, 1), lambda b: (0, 0)),
        ],
        out_shape=[
            jax.ShapeDtypeStruct((BATCH, DIM, PIX), jnp.float32),
            jax.ShapeDtypeStruct((BATCH, 1, PIX), jnp.int32),
            jax.ShapeDtypeStruct((1, 1), jnp.float32),
        ],
    )(z3, embed_w)
    z_q_st = zq3.reshape(z.shape)
    encoding_indices = idx3.reshape(BATCH, 32, 32)
    return z_q_st, loss.reshape(()), encoding_indices
